# baseline (speedup 1.0000x reference)
import jax
import jax.numpy as jnp
from jax import lax
from jax.experimental import pallas as pl
from jax.experimental.pallas import tpu as pltpu
from jax.experimental.pallas import tpu_sc as plsc

B, O, F, V, D = 1024, 50, 26, 100000, 64
H = D // 16
N = B * O * F
LN_EPS = 1e-5

FC = 2
NCHUNK = F // FC
NRC = FC * B * O
G = 16
GW = 8
CH = G * GW
NC, NS = 2, 16
NW = NC * NS
PER_W = NRC // NW
CPW = PER_W // CH
NB = 5
NSUP = CPW // NB


def _sc_gather_body(tab_ref, idx_ref, out_ref, idx_v, rows_v, gsem, wsem):
    wid = lax.axis_index("s") * NC + lax.axis_index("c")
    base_ch = wid * CPW
    pltpu.sync_copy(idx_ref.at[wid], idx_v)

    def start_gathers(j, b):
        for k in range(GW):
            ivec = idx_v[j, pl.ds(k * G, G)]
            pltpu.async_copy(tab_ref.at[ivec],
                             rows_v.at[b, pl.ds(k * G, G)], gsem.at[b])

    def wait_gathers(b):
        pltpu.make_async_copy(tab_ref.at[pl.ds(0, CH)], rows_v.at[b],
                              gsem.at[b]).wait()

    for b in range(NB):
        start_gathers(b, b)

    def super_step(g, carry):
        for b in range(NB):
            j = g * NB + b
            wait_gathers(b)
            pltpu.async_copy(rows_v.at[b],
                             out_ref.at[pl.ds((base_ch + j) * CH, CH)],
                             wsem.at[b])

            @pl.when(g < NSUP - 1)
            def _():
                pltpu.make_async_copy(
                    rows_v.at[b],
                    out_ref.at[pl.ds((base_ch + j) * CH, CH)],
                    wsem.at[b]).wait()
                start_gathers(j + NB, b)
        return carry

    lax.fori_loop(0, NSUP, super_step, 0)
    for b in range(NB):
        j = (NSUP - 1) * NB + b
        pltpu.make_async_copy(rows_v.at[b],
                              out_ref.at[pl.ds((base_ch + j) * CH, CH)],
                              wsem.at[b]).wait()


def _make_sc_gather():
    return pl.kernel(
        _sc_gather_body,
        out_type=jax.ShapeDtypeStruct((NRC, 2 * D), jnp.float32),
        mesh=plsc.VectorSubcoreMesh(core_axis_name="c", subcore_axis_name="s"),
        compiler_params=pltpu.CompilerParams(use_tc_tiling_on_sc=True),
        scratch_types=[
            pltpu.VMEM((CPW, 2 * D), jnp.int32),
            pltpu.VMEM((NB, CH, 2 * D), jnp.float32),
            pltpu.SemaphoreType.DMA((NB,)),
            pltpu.SemaphoreType.DMA((NB,)),
        ],
    )


def _slot_body(ctx_ref, w1_ref, b1_ref, w2_ref, b2_ref, w_ref):
    h = jnp.dot(ctx_ref[...], w1_ref[...], preferred_element_type=jnp.float32)
    h = h + b1_ref[...]
    sp = jnp.maximum(h, 0.0) + jnp.log(1.0 + jnp.exp(-jnp.abs(h)))
    m = h * jnp.tanh(sp)
    logits = jnp.dot(m, w2_ref[...], preferred_element_type=jnp.float32)
    logits = logits + b2_ref[...]
    mx = jnp.max(logits, axis=-1, keepdims=True)
    e = jnp.exp(logits - mx)
    w_ref[...] = e / jnp.sum(e, axis=-1, keepdims=True)


_slot = pl.pallas_call(
    _slot_body,
    out_shape=jax.ShapeDtypeStruct((B, F), jnp.float32),
)


RB = 400


def _mix_body(*refs):
    g_refs = refs[:NCHUNK]
    wexp_ref, lnw_ref, lnb_ref, avg_ref, out_ref = refs[NCHUNK:]
    wexp = wexp_ref[...]
    avg = avg_ref[...]
    acc = jnp.zeros((RB, D), jnp.float32)
    for k in range(NCHUNK):
        for j in range(FC):
            f = k * FC + j
            e = g_refs[k][j][:, 0:D]
            mu = jnp.dot(e, avg, preferred_element_type=jnp.float32)
            c = e - mu
            var = jnp.dot(c * c, avg, preferred_element_type=jnp.float32)
            nrm = c * lax.rsqrt(var + LN_EPS)
            sc = nrm * lnw_ref[f : f + 1, :] + lnb_ref[f : f + 1, :]
            acc = acc + wexp[:, f : f + 1] * sc
    out_ref[...] = acc


_mix = pl.pallas_call(
    _mix_body,
    grid=(B * O // RB,),
    in_specs=(
        [pl.BlockSpec((FC, RB, 2 * D), lambda i: (0, i, 0))] * NCHUNK
        + [
            pl.BlockSpec((RB, F), lambda i: (i, 0)),
            pl.BlockSpec((F, D), lambda i: (0, 0)),
            pl.BlockSpec((F, D), lambda i: (0, 0)),
            pl.BlockSpec((D, D), lambda i: (0, 0)),
        ]
    ),
    out_specs=pl.BlockSpec((RB, D), lambda i: (i, 0)),
    out_shape=jax.ShapeDtypeStruct((B * O, D), jnp.float32),
)


def kernel(context, object_map, tables, ln_w, ln_b, W1, b1, W2, b2):
    om_t = object_map.astype(jnp.int32).transpose(2, 0, 1).reshape(F, B * O)
    loc_off = (jnp.arange(FC, dtype=jnp.int32) * V)[:, None]

    sc_gather = _make_sc_gather()
    chunks = []
    for k in range(NCHUNK):
        idx_k = om_t[k * FC : (k + 1) * FC] + loc_off
        idx_k = idx_k.reshape(NW, CPW, 2 * D)
        tab_k = jnp.pad(tables[k * FC : (k + 1) * FC],
                        ((0, 0), (0, 0), (0, D))).reshape(FC * V, 2 * D)
        chunks.append(sc_gather(tab_k, idx_k).reshape(FC, B * O, 2 * D))

    w = _slot(context, W1, b1.reshape(1, H), W2, b2.reshape(1, F))
    wexp = jnp.repeat(w, O, axis=0)
    avg = jnp.full((D, D), 1.0 / D, dtype=jnp.float32)

    out = _mix(*chunks, wexp, ln_w, ln_b, avg)
    return out.reshape(B, O, D)


# speedup vs baseline: 1.1146x; 1.1146x over previous
import jax
import jax.numpy as jnp
from jax import lax
from jax.experimental import pallas as pl
from jax.experimental.pallas import tpu as pltpu
from jax.experimental.pallas import tpu_sc as plsc

B, O, F, V, D = 1024, 50, 26, 100000, 64
H = D // 16
N = B * O * F
LN_EPS = 1e-5

G = 16
GW = 8
CH = G * GW
NC, NS = 2, 16
NW = NC * NS
PER_W = N // NW
CPW = PER_W // CH
NB = 5
NSUP = CPW // NB


def _sc_gather_body(tab_ref, idx_ref, out_ref, idx_v, rows_v, gsem, wsem):
    wid = lax.axis_index("s") * NC + lax.axis_index("c")
    base_ch = wid * CPW
    pltpu.sync_copy(idx_ref.at[wid], idx_v)

    def start_gathers(j, b):
        for k in range(GW):
            ivec = idx_v[j, pl.ds(k * G, G)]
            pltpu.async_copy(tab_ref.at[ivec],
                             rows_v.at[b, pl.ds(k * G, G)], gsem.at[b])

    def wait_gathers(b):
        pltpu.make_async_copy(tab_ref.at[pl.ds(0, CH)], rows_v.at[b],
                              gsem.at[b]).wait()

    for b in range(NB):
        start_gathers(b, b)

    def super_step(g, carry):
        for b in range(NB):
            j = g * NB + b
            wait_gathers(b)
            pltpu.async_copy(rows_v.at[b],
                             out_ref.at[pl.ds((base_ch + j) * CH, CH)],
                             wsem.at[b])

            @pl.when(g < NSUP - 1)
            def _():
                pltpu.make_async_copy(
                    rows_v.at[b],
                    out_ref.at[pl.ds((base_ch + j) * CH, CH)],
                    wsem.at[b]).wait()
                start_gathers(j + NB, b)
        return carry

    lax.fori_loop(0, NSUP, super_step, 0)
    for b in range(NB):
        j = (NSUP - 1) * NB + b
        pltpu.make_async_copy(rows_v.at[b],
                              out_ref.at[pl.ds((base_ch + j) * CH, CH)],
                              wsem.at[b]).wait()


def _make_sc_gather():
    return pl.kernel(
        _sc_gather_body,
        out_type=jax.ShapeDtypeStruct((N, 2 * D), jnp.float32),
        mesh=plsc.VectorSubcoreMesh(core_axis_name="c", subcore_axis_name="s"),
        compiler_params=pltpu.CompilerParams(use_tc_tiling_on_sc=True),
        scratch_types=[
            pltpu.VMEM((CPW, 2 * D), jnp.int32),
            pltpu.VMEM((NB, CH, 2 * D), jnp.float32),
            pltpu.SemaphoreType.DMA((NB,)),
            pltpu.SemaphoreType.DMA((NB,)),
        ],
    )


BV = 5000


def _pad_body(t_ref, out_ref):
    out_ref[...] = jnp.concatenate(
        [t_ref[0], jnp.zeros((BV, D), jnp.float32)], axis=1)


_pad128 = pl.pallas_call(
    _pad_body,
    grid=(F, V // BV),
    in_specs=[pl.BlockSpec((1, BV, D), lambda f, v: (f, v, 0))],
    out_specs=pl.BlockSpec((BV, 2 * D), lambda f, v: (f * (V // BV) + v, 0)),
    out_shape=jax.ShapeDtypeStruct((F * V, 2 * D), jnp.float32),
)


def _slot_body(ctx_ref, w1_ref, b1_ref, w2_ref, b2_ref, w_ref):
    h = jnp.dot(ctx_ref[...], w1_ref[...], preferred_element_type=jnp.float32)
    h = h + b1_ref[...]
    sp = jnp.maximum(h, 0.0) + jnp.log(1.0 + jnp.exp(-jnp.abs(h)))
    m = h * jnp.tanh(sp)
    logits = jnp.dot(m, w2_ref[...], preferred_element_type=jnp.float32)
    logits = logits + b2_ref[...]
    mx = jnp.max(logits, axis=-1, keepdims=True)
    e = jnp.exp(logits - mx)
    w_ref[...] = e / jnp.sum(e, axis=-1, keepdims=True)


_slot = pl.pallas_call(
    _slot_body,
    out_shape=jax.ShapeDtypeStruct((B, F), jnp.float32),
)


RB = 400


def _mix_body(g_ref, wexp_ref, lnw_ref, lnb_ref, avg_ref, out_ref):
    wexp = wexp_ref[...]
    avg = avg_ref[...]
    acc = jnp.zeros((RB, D), jnp.float32)
    for f in range(F):
        e = g_ref[f][:, 0:D]
        mu = jnp.dot(e, avg, preferred_element_type=jnp.float32)
        c = e - mu
        var = jnp.dot(c * c, avg, preferred_element_type=jnp.float32)
        nrm = c * lax.rsqrt(var + LN_EPS)
        sc = nrm * lnw_ref[f : f + 1, :] + lnb_ref[f : f + 1, :]
        acc = acc + wexp[:, f : f + 1] * sc
    out_ref[...] = acc


_mix = pl.pallas_call(
    _mix_body,
    grid=(B * O // RB,),
    in_specs=[
        pl.BlockSpec((F, RB, 2 * D), lambda i: (0, i, 0)),
        pl.BlockSpec((RB, F), lambda i: (i, 0)),
        pl.BlockSpec((F, D), lambda i: (0, 0)),
        pl.BlockSpec((F, D), lambda i: (0, 0)),
        pl.BlockSpec((D, D), lambda i: (0, 0)),
    ],
    out_specs=pl.BlockSpec((RB, D), lambda i: (i, 0)),
    out_shape=jax.ShapeDtypeStruct((B * O, D), jnp.float32),
)


def kernel(context, object_map, tables, ln_w, ln_b, W1, b1, W2, b2):
    om = object_map.astype(jnp.int32)
    idx = om.transpose(2, 0, 1).reshape(F, B * O)
    idx = idx + (jnp.arange(F, dtype=jnp.int32) * V)[:, None]
    idx3d = idx.reshape(NW, CPW, 2 * D)

    tab_pad = _pad128(tables)
    gathered = _make_sc_gather()(tab_pad, idx3d)

    w = _slot(context, W1, b1.reshape(1, H), W2, b2.reshape(1, F))
    wexp = jnp.repeat(w, O, axis=0)
    avg = jnp.full((D, D), 1.0 / D, dtype=jnp.float32)

    out = _mix(gathered.reshape(F, B * O, 2 * D), wexp, ln_w, ln_b, avg)
    return out.reshape(B, O, D)


# speedup vs baseline: 1.3542x; 1.2150x over previous
import jax
import jax.numpy as jnp
from jax import lax
from jax.experimental import pallas as pl
from jax.experimental.pallas import tpu as pltpu
from jax.experimental.pallas import tpu_sc as plsc

B, O, F, V, D = 1024, 50, 26, 100000, 64
H = D // 16
N = B * O * F
LN_EPS = 1e-5

G = 16
GW = 8
CH = G * GW
NC, NS = 2, 16
NW = NC * NS
PER_W = N // NW
CPW = PER_W // CH
NB = 5
NSUP = CPW // NB


def _sc_gather_body(tab_ref, idx_ref, out_ref, idx_v, rows_v, gsem, wsem):
    wid = lax.axis_index("s") * NC + lax.axis_index("c")
    base_ch = wid * CPW
    pltpu.sync_copy(idx_ref.at[wid], idx_v)

    def start_gathers(j, b):
        for k in range(GW):
            ivec = idx_v[j, pl.ds(k * G, G)]
            pltpu.async_copy(tab_ref.at[ivec],
                             rows_v.at[b, pl.ds(k * G, G)], gsem.at[b])

    def wait_gathers(b):
        pltpu.make_async_copy(tab_ref.at[pl.ds(0, CH)], rows_v.at[b],
                              gsem.at[b]).wait()

    for b in range(NB):
        start_gathers(b, b)

    def super_step(g, carry):
        for b in range(NB):
            j = g * NB + b
            wait_gathers(b)
            pltpu.async_copy(rows_v.at[b],
                             out_ref.at[pl.ds((base_ch + j) * CH, CH)],
                             wsem.at[b])

            @pl.when(g < NSUP - 1)
            def _():
                pltpu.make_async_copy(
                    rows_v.at[b],
                    out_ref.at[pl.ds((base_ch + j) * CH, CH)],
                    wsem.at[b]).wait()
                start_gathers(j + NB, b)
        return carry

    lax.fori_loop(0, NSUP, super_step, 0)
    for b in range(NB):
        j = (NSUP - 1) * NB + b
        pltpu.make_async_copy(rows_v.at[b],
                              out_ref.at[pl.ds((base_ch + j) * CH, CH)],
                              wsem.at[b]).wait()


def _make_sc_gather():
    return pl.kernel(
        _sc_gather_body,
        out_type=jax.ShapeDtypeStruct((N, 2 * D), jnp.float32),
        mesh=plsc.VectorSubcoreMesh(core_axis_name="c", subcore_axis_name="s"),
        compiler_params=pltpu.CompilerParams(use_tc_tiling_on_sc=True),
        scratch_types=[
            pltpu.VMEM((CPW, 2 * D), jnp.int32),
            pltpu.VMEM((NB, CH, 2 * D), jnp.float32),
            pltpu.SemaphoreType.DMA((NB,)),
            pltpu.SemaphoreType.DMA((NB,)),
        ],
    )


def _slot_body(ctx_ref, w1_ref, b1_ref, w2_ref, b2_ref, w_ref):
    h = jnp.dot(ctx_ref[...], w1_ref[...], preferred_element_type=jnp.float32)
    h = h + b1_ref[...]
    sp = jnp.maximum(h, 0.0) + jnp.log(1.0 + jnp.exp(-jnp.abs(h)))
    m = h * jnp.tanh(sp)
    logits = jnp.dot(m, w2_ref[...], preferred_element_type=jnp.float32)
    logits = logits + b2_ref[...]
    mx = jnp.max(logits, axis=-1, keepdims=True)
    e = jnp.exp(logits - mx)
    w_ref[...] = e / jnp.sum(e, axis=-1, keepdims=True)


_slot = pl.pallas_call(
    _slot_body,
    out_shape=jax.ShapeDtypeStruct((B, F), jnp.float32),
)


RB = 400


def _mix_body(g_ref, wexp_ref, lnw_ref, lnb_ref, avg_ref, out_ref):
    wexp = wexp_ref[...]
    avg = avg_ref[...]
    acc = jnp.zeros((RB, D), jnp.float32)
    for f in range(F):
        e = g_ref[f][:, 0:D]
        mu = jnp.dot(e, avg, preferred_element_type=jnp.float32)
        c = e - mu
        var = jnp.dot(c * c, avg, preferred_element_type=jnp.float32)
        nrm = c * lax.rsqrt(var + LN_EPS)
        sc = nrm * lnw_ref[f : f + 1, :] + lnb_ref[f : f + 1, :]
        acc = acc + wexp[:, f : f + 1] * sc
    out_ref[...] = acc


_mix = pl.pallas_call(
    _mix_body,
    grid=(B * O // RB,),
    in_specs=[
        pl.BlockSpec((F, RB, 2 * D), lambda i: (0, i, 0)),
        pl.BlockSpec((RB, F), lambda i: (i, 0)),
        pl.BlockSpec((F, D), lambda i: (0, 0)),
        pl.BlockSpec((F, D), lambda i: (0, 0)),
        pl.BlockSpec((D, D), lambda i: (0, 0)),
    ],
    out_specs=pl.BlockSpec((RB, D), lambda i: (i, 0)),
    out_shape=jax.ShapeDtypeStruct((B * O, D), jnp.float32),
)


def kernel(context, object_map, tables, ln_w, ln_b, W1, b1, W2, b2):
    om = object_map.astype(jnp.int32)
    idx = om.transpose(2, 0, 1).reshape(F, B * O)
    idx = idx + (jnp.arange(F, dtype=jnp.int32) * V)[:, None]
    idx3d = idx.reshape(NW, CPW, 2 * D)

    tab_pad = jnp.pad(tables, ((0, 0), (0, 0), (0, D))).reshape(F * V, 2 * D)
    gathered = _make_sc_gather()(tab_pad, idx3d)

    w = _slot(context, W1, b1.reshape(1, H), W2, b2.reshape(1, F))
    wexp = jnp.repeat(w, O, axis=0)
    avg = jnp.full((D, D), 1.0 / D, dtype=jnp.float32)

    out = _mix(gathered.reshape(F, B * O, 2 * D), wexp, ln_w, ln_b, avg)
    return out.reshape(B, O, D)


# speedup vs baseline: 1.3552x; 1.0008x over previous
import jax
import jax.numpy as jnp
from jax import lax
from jax.experimental import pallas as pl
from jax.experimental.pallas import tpu as pltpu
from jax.experimental.pallas import tpu_sc as plsc

B, O, F, V, D = 1024, 50, 26, 100000, 64
H = D // 16
N = B * O * F
LN_EPS = 1e-5

G = 16
GW = 8
CH = G * GW
NC, NS = 2, 16
NW = NC * NS
PER_W = N // NW
CPW = PER_W // CH
NB = 5
NSUP = CPW // NB


def _sc_gather_body(tab_ref, idx_ref, out_ref, idx_v, rows_v, gsem, wsem):
    wid = lax.axis_index("s") * NC + lax.axis_index("c")
    base_ch = wid * CPW
    pltpu.sync_copy(idx_ref.at[wid], idx_v)

    def start_gathers(j, b):
        for k in range(GW):
            ivec = idx_v[j, pl.ds(k * G, G)]
            pltpu.async_copy(tab_ref.at[ivec],
                             rows_v.at[b, pl.ds(k * G, G)], gsem.at[b])

    def wait_gathers(b):
        pltpu.make_async_copy(tab_ref.at[pl.ds(0, CH)], rows_v.at[b],
                              gsem.at[b]).wait()

    for b in range(NB):
        start_gathers(b, b)

    def super_step(g, carry):
        for b in range(NB):
            j = g * NB + b
            wait_gathers(b)
            pltpu.async_copy(rows_v.at[b],
                             out_ref.at[pl.ds((base_ch + j) * CH, CH)],
                             wsem.at[b])

            @pl.when(g < NSUP - 1)
            def _():
                pltpu.make_async_copy(
                    rows_v.at[b],
                    out_ref.at[pl.ds((base_ch + j) * CH, CH)],
                    wsem.at[b]).wait()
                start_gathers(j + NB, b)
        return carry

    lax.fori_loop(0, NSUP, super_step, 0)
    for b in range(NB):
        j = (NSUP - 1) * NB + b
        pltpu.make_async_copy(rows_v.at[b],
                              out_ref.at[pl.ds((base_ch + j) * CH, CH)],
                              wsem.at[b]).wait()


def _make_sc_gather():
    return pl.kernel(
        _sc_gather_body,
        out_type=jax.ShapeDtypeStruct((N, 2 * D), jnp.float32),
        mesh=plsc.VectorSubcoreMesh(core_axis_name="c", subcore_axis_name="s"),
        compiler_params=pltpu.CompilerParams(use_tc_tiling_on_sc=True),
        scratch_types=[
            pltpu.VMEM((CPW, 2 * D), jnp.int32),
            pltpu.VMEM((NB, CH, 2 * D), jnp.float32),
            pltpu.SemaphoreType.DMA((NB,)),
            pltpu.SemaphoreType.DMA((NB,)),
        ],
    )


def _slot_body(ctx_ref, w1_ref, b1_ref, w2_ref, b2_ref, w_ref):
    h = jnp.dot(ctx_ref[...], w1_ref[...], preferred_element_type=jnp.float32)
    h = h + b1_ref[...]
    sp = jnp.maximum(h, 0.0) + jnp.log(1.0 + jnp.exp(-jnp.abs(h)))
    m = h * jnp.tanh(sp)
    logits = jnp.dot(m, w2_ref[...], preferred_element_type=jnp.float32)
    logits = logits + b2_ref[...]
    mx = jnp.max(logits, axis=-1, keepdims=True)
    e = jnp.exp(logits - mx)
    w_ref[...] = e / jnp.sum(e, axis=-1, keepdims=True)


_slot = pl.pallas_call(
    _slot_body,
    out_shape=jax.ShapeDtypeStruct((B, F), jnp.float32),
)


RB = 400


def _mix_body(g_ref, wexp_ref, lnw_ref, lnb_ref, avg_ref, out_ref):
    wexp = wexp_ref[...]
    avg = avg_ref[...]
    acc = jnp.zeros((RB, D), jnp.float32)
    for f in range(F):
        e = g_ref[f][:, 0:D]
        mu = jnp.dot(e, avg, preferred_element_type=jnp.float32)
        c = e - mu
        var = jnp.dot(c * c, avg, preferred_element_type=jnp.float32)
        nrm = c * lax.rsqrt(var + LN_EPS)
        sc = nrm * lnw_ref[f : f + 1, :] + lnb_ref[f : f + 1, :]
        acc = acc + wexp[:, f : f + 1] * sc
    out_ref[...] = acc


_mix = pl.pallas_call(
    _mix_body,
    grid=(B * O // RB,),
    in_specs=[
        pl.BlockSpec((F, RB, 2 * D), lambda i: (0, i, 0)),
        pl.BlockSpec((RB, F), lambda i: (i, 0)),
        pl.BlockSpec((F, D), lambda i: (0, 0)),
        pl.BlockSpec((F, D), lambda i: (0, 0)),
        pl.BlockSpec((D, D), lambda i: (0, 0)),
    ],
    out_specs=pl.BlockSpec((RB, D), lambda i: (i, 0)),
    out_shape=jax.ShapeDtypeStruct((B * O, D), jnp.float32),
)


def kernel(context, object_map, tables, ln_w, ln_b, W1, b1, W2, b2):
    om = object_map.astype(jnp.int32)
    idx = om.transpose(2, 0, 1).reshape(F, B * O)
    idx = idx + (jnp.arange(F, dtype=jnp.int32) * V)[:, None]
    idx3d = idx.reshape(NW, CPW, 2 * D)

    tab_pad = jnp.concatenate(
        [tables, jnp.zeros_like(tables)], axis=-1).reshape(F * V, 2 * D)
    gathered = _make_sc_gather()(tab_pad, idx3d)

    w = _slot(context, W1, b1.reshape(1, H), W2, b2.reshape(1, F))
    wexp = jnp.repeat(w, O, axis=0)
    avg = jnp.full((D, D), 1.0 / D, dtype=jnp.float32)

    out = _mix(gathered.reshape(F, B * O, 2 * D), wexp, ln_w, ln_b, avg)
    return out.reshape(B, O, D)
